# baseline (speedup 1.0000x reference)
import functools

import jax
import jax.numpy as jnp
from jax import lax
from jax.experimental import pallas as pl
from jax.experimental.pallas import tpu as pltpu
from jax.experimental.pallas import tpu_sc as plsc

_TOP_K = 4
_NC = 2
_NS = 16
_NW = _NC * _NS


def _topk_body(q_ref, keys_any, idx_ref, knorm_v, sem):
    num_prompts = knorm_v.shape[0]

    @pl.when(pl.program_id(0) == 0)
    def _init():
        copy = pltpu.make_async_copy(keys_any, knorm_v, sem)
        copy.start()
        copy.wait()
        chunk = 1024
        for c in range(num_prompts // chunk):
            sl = pl.ds(c * chunk, chunk)
            blk = knorm_v[sl, :]
            nrm = jnp.sqrt(jnp.sum(blk * blk, axis=1, keepdims=True))
            knorm_v[sl, :] = blk / jnp.maximum(nrm, 1e-12)

    q = q_ref[...]
    qnrm = jnp.sqrt(jnp.sum(q * q, axis=1, keepdims=True))
    q = q / jnp.maximum(qnrm, 1e-12)
    kn = knorm_v[...]
    s = lax.dot_general(q, kn, (((1,), (1,)), ((), ())),
                        preferred_element_type=jnp.float32)
    iota = lax.broadcasted_iota(jnp.int32, s.shape, 1)
    cols = []
    for _ in range(_TOP_K):
        m = jnp.max(s, axis=1, keepdims=True)
        idx_t = jnp.min(jnp.where(s == m, iota, num_prompts), axis=1,
                        keepdims=True)
        cols.append(idx_t)
        s = jnp.where(iota == idx_t, -jnp.inf, s)
    idx_ref[...] = jnp.concatenate(cols, axis=1)


def _topk_tc(query_features, prompt_keys, interpret=False):
    B, D = query_features.shape
    NP = prompt_keys.shape[0]
    QB = 128
    return pl.pallas_call(
        _topk_body,
        grid=(B // QB,),
        in_specs=[
            pl.BlockSpec((QB, D), lambda i: (i, 0)),
            pl.BlockSpec(memory_space=pltpu.MemorySpace.HBM),
        ],
        out_specs=pl.BlockSpec((QB, _TOP_K), lambda i: (i, 0)),
        out_shape=jax.ShapeDtypeStruct((B, _TOP_K), jnp.int32),
        scratch_shapes=[
            pltpu.VMEM((NP, D), jnp.float32),
            pltpu.SemaphoreType.DMA,
        ],
        compiler_params=pltpu.CompilerParams(
            dimension_semantics=("arbitrary",),
        ),
        interpret=interpret,
    )(query_features, prompt_keys)


def _gather_sc(prompts, idx, out_ref, qoff):
    NP, PLen, D = prompts.shape
    BH, K = idx.shape
    qpw = BH // _NW
    half = K // 2
    nch = qpw * 2
    nbuf = 4
    mesh = plsc.VectorSubcoreMesh(core_axis_name="c", subcore_axis_name="s")

    @functools.partial(
        pl.kernel,
        mesh=mesh,
        out_type=(),
        scratch_types=[
            pltpu.VMEM((qpw, K), jnp.int32),
            [pltpu.VMEM((half, PLen, D), jnp.float32)] * nbuf,
            [pltpu.SemaphoreType.DMA] * nbuf,
            [pltpu.SemaphoreType.DMA] * nbuf,
        ],
    )
    def k(prompts_hbm, idx_hbm, out_hbm, idx_v, bufs, gsems, wsems):
        wid = lax.axis_index("s") * _NC + lax.axis_index("c")
        qbase = qoff + wid * qpw
        pltpu.sync_copy(idx_hbm.at[pl.ds(wid * qpw, qpw)], idx_v)

        def g_copy(ch, i):
            return pltpu.make_async_copy(
                prompts_hbm.at[idx_v.at[ch // 2, pl.ds((ch % 2) * half, half)]],
                bufs[i], gsems[i])

        def w_copies(ch, i):
            return [
                pltpu.make_async_copy(
                    bufs[i],
                    out_hbm.at[qbase + ch // 2, pl.ds((ch % 2) * half, half)],
                    wsems[i])
            ]

        for c in range(nbuf - 1):
            g_copy(c, c).start()

        def body(j, carry):
            for i in range(nbuf):
                ch = nbuf * j + i
                g_copy(ch, i).wait()
                for c in w_copies(ch, i):
                    c.start()
                prev = ch - 1
                if i == 0:
                    @pl.when(j > 0)
                    def _():
                        for c in w_copies(prev, (nbuf - 1)):
                            c.wait()
                else:
                    for c in w_copies(prev, i - 1):
                        c.wait()
                nxt = ch + nbuf - 1
                if i == 0:
                    g_copy(nxt, nbuf - 1).start()
                else:
                    @pl.when(j < nch // nbuf - 1)
                    def _():
                        g_copy(nxt, i - 1).start()
            return carry

        lax.fori_loop(0, nch // nbuf, body, 0)
        for c in w_copies(nch - 1, nbuf - 1):
            c.wait()

    k(prompts, idx, out_ref)


def kernel(query_features, prompts, prompt_keys, top_k):
    B, D = query_features.shape
    NP, PLen, _ = prompts.shape
    nsplit = 4
    H = B // nsplit
    out_ref = jax.empty_ref(
        jax.ShapeDtypeStruct((B, _TOP_K, PLen, D), jnp.float32))
    for p in range(nsplit):
        idx_p = _topk_tc(query_features[p * H:(p + 1) * H], prompt_keys)
        _gather_sc(prompts, idx_p, out_ref, p * H)
    return out_ref[...].reshape(B, _TOP_K * PLen, D)


# speedup vs baseline: 1.0243x; 1.0243x over previous
import functools

import jax
import jax.numpy as jnp
from jax import lax
from jax.experimental import pallas as pl
from jax.experimental.pallas import tpu as pltpu
from jax.experimental.pallas import tpu_sc as plsc

_TOP_K = 4
_NC = 2
_NS = 16
_NW = _NC * _NS


def _topk_body(q_ref, keys_any, idx_ref, knorm_v, sem):
    num_prompts = knorm_v.shape[0]

    @pl.when(pl.program_id(0) == 0)
    def _init():
        copy = pltpu.make_async_copy(keys_any, knorm_v, sem)
        copy.start()
        copy.wait()
        chunk = 1024
        for c in range(num_prompts // chunk):
            sl = pl.ds(c * chunk, chunk)
            blk = knorm_v[sl, :]
            nrm = jnp.sqrt(jnp.sum(blk * blk, axis=1, keepdims=True))
            knorm_v[sl, :] = blk / jnp.maximum(nrm, 1e-12)

    q = q_ref[...]
    qnrm = jnp.sqrt(jnp.sum(q * q, axis=1, keepdims=True))
    q = q / jnp.maximum(qnrm, 1e-12)
    kn = knorm_v[...]
    s = lax.dot_general(q, kn, (((1,), (1,)), ((), ())),
                        preferred_element_type=jnp.float32)
    iota = lax.broadcasted_iota(jnp.int32, s.shape, 1)
    cols = []
    for _ in range(_TOP_K):
        m = jnp.max(s, axis=1, keepdims=True)
        idx_t = jnp.min(jnp.where(s == m, iota, num_prompts), axis=1,
                        keepdims=True)
        cols.append(idx_t)
        s = jnp.where(iota == idx_t, -jnp.inf, s)
    idx_ref[...] = jnp.concatenate(cols, axis=1)


def _topk_tc(query_features, prompt_keys, interpret=False):
    B, D = query_features.shape
    NP = prompt_keys.shape[0]
    QB = 128
    return pl.pallas_call(
        _topk_body,
        grid=(B // QB,),
        in_specs=[
            pl.BlockSpec((QB, D), lambda i: (i, 0)),
            pl.BlockSpec(memory_space=pltpu.MemorySpace.HBM),
        ],
        out_specs=pl.BlockSpec((QB, _TOP_K), lambda i: (i, 0)),
        out_shape=jax.ShapeDtypeStruct((B, _TOP_K), jnp.int32),
        scratch_shapes=[
            pltpu.VMEM((NP, D), jnp.float32),
            pltpu.SemaphoreType.DMA,
        ],
        compiler_params=pltpu.CompilerParams(
            dimension_semantics=("arbitrary",),
        ),
        interpret=interpret,
    )(query_features, prompt_keys)


def _gather_sc(prompts, idx, out_ref, qoff):
    NP, PLen, D = prompts.shape
    BH, K = idx.shape
    qpw = BH // _NW
    nch = qpw * K
    nbuf = 8
    depth = 4
    mesh = plsc.VectorSubcoreMesh(core_axis_name="c", subcore_axis_name="s")

    @functools.partial(
        pl.kernel,
        mesh=mesh,
        out_type=(),
        scratch_types=[
            pltpu.VMEM((qpw, K), jnp.int32),
            [pltpu.VMEM((1, PLen, D), jnp.float32)] * nbuf,
            [pltpu.SemaphoreType.DMA] * nbuf,
            [pltpu.SemaphoreType.DMA] * nbuf,
        ],
    )
    def k(prompts_hbm, idx_hbm, out_hbm, idx_v, bufs, gsems, wsems):
        wid = lax.axis_index("s") * _NC + lax.axis_index("c")
        qbase = qoff + wid * qpw
        pltpu.sync_copy(idx_hbm.at[pl.ds(wid * qpw, qpw)], idx_v)

        def g_copy(ch, i):
            return pltpu.make_async_copy(
                prompts_hbm.at[idx_v.at[ch // K, pl.ds(ch % K, 1)]],
                bufs[i], gsems[i])

        def w_copy(ch, i):
            return pltpu.make_async_copy(
                bufs[i],
                out_hbm.at[qbase + ch // K, pl.ds(ch % K, 1)],
                wsems[i])

        for c in range(depth):
            g_copy(c, c).start()

        def body(j, carry):
            for i in range(nbuf):
                ch = nbuf * j + i
                g_copy(ch, i).wait()
                w_copy(ch, i).start()
                prev = ch - depth
                if i < depth:
                    @pl.when(j > 0)
                    def _():
                        w_copy(prev, (i - depth) % nbuf).wait()
                else:
                    w_copy(prev, i - depth).wait()
                nxt = ch + depth
                if i < nbuf - depth:
                    g_copy(nxt, (i + depth) % nbuf).start()
                else:
                    @pl.when(j < nch // nbuf - 1)
                    def _():
                        g_copy(nxt, i - depth).start()
            return carry

        lax.fori_loop(0, nch // nbuf, body, 0)
        for c in range(depth):
            w_copy(nch - depth + c, (nch - depth + c) % nbuf).wait()

    k(prompts, idx, out_ref)


def kernel(query_features, prompts, prompt_keys, top_k):
    B, D = query_features.shape
    NP, PLen, _ = prompts.shape
    nsplit = 2
    H = B // nsplit
    out_ref = jax.empty_ref(
        jax.ShapeDtypeStruct((B, _TOP_K, PLen, D), jnp.float32))
    for p in range(nsplit):
        idx_p = _topk_tc(query_features[p * H:(p + 1) * H], prompt_keys)
        _gather_sc(prompts, idx_p, out_ref, p * H)
    return out_ref[...].reshape(B, _TOP_K * PLen, D)
